# baseline (speedup 1.0000x reference)
import functools

import jax
import jax.numpy as jnp
from jax import lax
from jax.experimental import pallas as pl
from jax.experimental.pallas import tpu as pltpu
from jax.experimental.pallas import tpu_sc as plsc

_N = 10000
_C = 128
_H = 4
_E = 320000

_NC = 2
_NS = 16
_NW = _NC * _NS

_BLK = 128
_NBLK = _E // _BLK
_CB = 20
_NTASK = _NBLK // _CB
_EPT = _CB * _BLK
_OPT = _EPT * _H
_KMAX = (_NTASK + _NW - 1) // _NW
_MITER = _EPT // 16


_NODE_BLK = 128
_NTBLK = (_N + _NODE_BLK - 1) // _NODE_BLK
_N_PAD = _NTBLK * _NODE_BLK
_MMB = 2048


_XA = 4992
_XB2 = _N - _XA


def _mm_body(w_ref, x_hbm, o_ref, xa, xb, s1, s2):
    cp1 = pltpu.make_async_copy(x_hbm.at[pl.ds(0, _XA)], xa, s1)
    cp1.start()
    cp2 = pltpu.make_async_copy(x_hbm.at[pl.ds(_XA, _XB2)], xb, s2)
    cp2.start()
    dims = (((0,), (1,)), ((), ()))
    cp1.wait()
    o_ref[0:_H, 0:_XA] = lax.dot_general(
        w_ref[0:_C, :], xa[...], dims, preferred_element_type=jnp.float32)
    o_ref[_H:2 * _H, 0:_XA] = lax.dot_general(
        w_ref[_C:2 * _C, :], xa[...], dims, preferred_element_type=jnp.float32)
    cp2.wait()
    o_ref[0:_H, _XA:_N] = lax.dot_general(
        w_ref[0:_C, :], xb[...], dims, preferred_element_type=jnp.float32)
    o_ref[_H:2 * _H, _XA:_N] = lax.dot_general(
        w_ref[_C:2 * _C, :], xb[...], dims, preferred_element_type=jnp.float32)


def _node_projection(x, att):
    return pl.pallas_call(
        _mm_body,
        in_specs=[
            pl.BlockSpec((2 * _C, _H), lambda: (0, 0)),
            pl.BlockSpec(memory_space=pltpu.HBM),
        ],
        out_specs=pl.BlockSpec((2 * _H, _N_PAD), lambda: (0, 0)),
        out_shape=jax.ShapeDtypeStruct((2 * _H, _N_PAD), jnp.float32),
        scratch_shapes=[
            pltpu.VMEM((_XA, _C), jnp.float32),
            pltpu.VMEM((_XB2, _C), jnp.float32),
            pltpu.SemaphoreType.DMA,
            pltpu.SemaphoreType.DMA,
        ],
    )(att, x)


_mesh = plsc.VectorSubcoreMesh(core_axis_name="c", subcore_axis_name="s",
                               num_cores=_NC, num_subcores=_NS)


@functools.partial(
    pl.kernel,
    out_type=jax.ShapeDtypeStruct((_E * _H,), jnp.float32),
    mesh=_mesh,
    compiler_params=pltpu.CompilerParams(needs_layout_passes=False),
    scratch_types=[
        pltpu.VMEM((_NTBLK * 2 * _H * _NODE_BLK,), jnp.float32),
        pltpu.VMEM_SHARED((_NTBLK * 2 * _H * _NODE_BLK,), jnp.float32),

        pltpu.VMEM((_EPT,), jnp.int32),
        pltpu.VMEM((_EPT,), jnp.int32),
        pltpu.VMEM((_EPT,), jnp.int32),
        pltpu.VMEM((_EPT,), jnp.int32),
        pltpu.VMEM((_OPT,), jnp.float32),
        pltpu.VMEM((_OPT,), jnp.float32),
        pltpu.SemaphoreType.DMA,
        pltpu.SemaphoreType.DMA,
        pltpu.SemaphoreType.DMA,
        pltpu.SemaphoreType.DMA,
    ],
)
def _sc_edges(tbl_hbm, adj_hbm, out_hbm, tbl_v, tbl_spm,
              src_v0, src_v1, tgt_v0, tgt_v1, out_v0, out_v1,
              sin0, sin1, sout0, sout1):
    wid = lax.axis_index("s") * _NC + lax.axis_index("c")

    srcs = (src_v0, src_v1)
    tgts = (tgt_v0, tgt_v1)
    outs = (out_v0, out_v1)
    sins = (sin0, sin1)
    souts = (sout0, sout1)

    def task_id(k):
        return wid + _NW * k

    def task_valid(k):
        return task_id(k) < _NTASK

    def start_in(k, b):
        off = task_id(k) * _EPT
        pltpu.async_copy(adj_hbm.at[0, pl.ds(off, _EPT)], srcs[b], sins[b])
        pltpu.async_copy(adj_hbm.at[1, pl.ds(off, _EPT)], tgts[b], sins[b])

    def wait_in(k, b):
        off = task_id(k) * _EPT
        pltpu.make_async_copy(adj_hbm.at[0, pl.ds(off, _EPT)], srcs[b],
                              sins[b]).wait()
        pltpu.make_async_copy(adj_hbm.at[1, pl.ds(off, _EPT)], tgts[b],
                              sins[b]).wait()

    def start_out(k, b):
        off = task_id(k) * _OPT
        pltpu.async_copy(outs[b], out_hbm.at[pl.ds(off, _OPT)], souts[b])

    def wait_out(k, b):
        off = task_id(k) * _OPT
        pltpu.make_async_copy(outs[b], out_hbm.at[pl.ds(off, _OPT)],
                              souts[b]).wait()

    def compute(k, b):
        src_v, tgt_v, out_v = srcs[b], tgts[b], outs[b]

        @plsc.parallel_loop(0, _MITER, unroll=4)
        def body(m):
            s = src_v[pl.ds(m * 16, 16)]
            t = tgt_v[pl.ds(m * 16, 16)]
            bs = (lax.shift_right_logical(s, 7) * (2 * _H * _NODE_BLK)
                  + lax.bitwise_and(s, _NODE_BLK - 1))
            bt = (lax.shift_right_logical(t, 7) * (2 * _H * _NODE_BLK)
                  + lax.bitwise_and(t, _NODE_BLK - 1))
            obase = (m // 8) * (_H * _BLK) + (m % 8) * 16
            for h in range(_H):
                u = plsc.load_gather(tbl_v, [bs + (h * _NODE_BLK)])
                v = plsc.load_gather(tbl_v, [bt + ((_H + h) * _NODE_BLK)])
                out_v[pl.ds(obase + h * _BLK, 16)] = jnp.maximum(u + v, 0.0)

    start_in(0, 0)
    _TBLW = _NTBLK * 2 * _H * _NODE_BLK
    _SEG = _TBLW // _NS
    sid = lax.axis_index("s")
    pltpu.sync_copy(tbl_hbm.at[pl.ds(sid * _SEG, _SEG)],
                    out_v0.at[pl.ds(0, _SEG)])
    pltpu.sync_copy(out_v0.at[pl.ds(0, _SEG)],
                    tbl_spm.at[pl.ds(sid * _SEG, _SEG)])
    plsc.subcore_barrier()
    pltpu.sync_copy(tbl_spm, tbl_v)

    def pair(kk, carry):
        k0 = 2 * kk
        k1 = k0 + 1

        @pl.when(task_valid(k1))
        def _():
            start_in(k1, 1)
        wait_in(k0, 0)

        @pl.when(k0 >= 2)
        def _():
            wait_out(k0 - 2, 0)
        compute(k0, 0)
        start_out(k0, 0)

        @pl.when(task_valid(k0 + 2))
        def _():
            start_in(k0 + 2, 0)

        @pl.when(task_valid(k1))
        def _():
            wait_in(k1, 1)

            @pl.when(k1 >= 3)
            def _():
                wait_out(k1 - 2, 1)
            compute(k1, 1)
            start_out(k1, 1)
        return carry

    lax.fori_loop(0, _KMAX // 2, pair, 0)
    wait_out(_KMAX - 2, 0)

    @pl.when(task_valid(_KMAX - 1))
    def _():
        wait_out(_KMAX - 1, 1)


def kernel(x_0, adjacency_0, att_parameter):
    adj = adjacency_0.astype(jnp.int32)
    p = _node_projection(x_0, att_parameter)
    tbl = p.reshape(2 * _H, _NTBLK, _NODE_BLK).transpose(1, 0, 2).reshape(-1)
    out_flat = _sc_edges(tbl, adj)
    out = out_flat.reshape(_NBLK, _H, _BLK).transpose(0, 2, 1).reshape(_E, _H)
    return out


# speedup vs baseline: 1.0134x; 1.0134x over previous
import functools

import jax
import jax.numpy as jnp
from jax import lax
from jax.experimental import pallas as pl
from jax.experimental.pallas import tpu as pltpu
from jax.experimental.pallas import tpu_sc as plsc

_N = 10000
_C = 128
_H = 4
_E = 320000

_NC = 2
_NS = 16
_NW = _NC * _NS

_BLK = 128
_NBLK = _E // _BLK
_CB = 20
_NTASK = _NBLK // _CB
_EPT = _CB * _BLK
_OPT = _EPT * _H
_KMAX = (_NTASK + _NW - 1) // _NW
_MITER = _EPT // 16


_NODE_BLK = 128
_NTBLK = (_N + _NODE_BLK - 1) // _NODE_BLK
_N_PAD = _NTBLK * _NODE_BLK
_MMB = 2048


def _mm_body(w_ref, x_ref, o_ref):
    xb = x_ref[...]
    dims = (((0,), (1,)), ((), ()))
    o_ref[0:_H, 0:_N] = lax.dot_general(
        w_ref[0:_C, :], xb, dims, preferred_element_type=jnp.float32)
    o_ref[_H:2 * _H, 0:_N] = lax.dot_general(
        w_ref[_C:2 * _C, :], xb, dims, preferred_element_type=jnp.float32)


def _node_projection(x, att):
    return pl.pallas_call(
        _mm_body,
        out_shape=jax.ShapeDtypeStruct((2 * _H, _N_PAD), jnp.float32),
    )(att, x)


_mesh = plsc.VectorSubcoreMesh(core_axis_name="c", subcore_axis_name="s",
                               num_cores=_NC, num_subcores=_NS)


@functools.partial(
    pl.kernel,
    out_type=jax.ShapeDtypeStruct((_E * _H,), jnp.float32),
    mesh=_mesh,
    compiler_params=pltpu.CompilerParams(needs_layout_passes=False),
    scratch_types=[
        pltpu.VMEM((_NTBLK * 2 * _H * _NODE_BLK,), jnp.float32),
        pltpu.VMEM_SHARED((_NTBLK * 2 * _H * _NODE_BLK,), jnp.float32),

        pltpu.VMEM((_EPT,), jnp.int32),
        pltpu.VMEM((_EPT,), jnp.int32),
        pltpu.VMEM((_EPT,), jnp.int32),
        pltpu.VMEM((_EPT,), jnp.int32),
        pltpu.VMEM((_OPT,), jnp.float32),
        pltpu.VMEM((_OPT,), jnp.float32),
        pltpu.SemaphoreType.DMA,
        pltpu.SemaphoreType.DMA,
        pltpu.SemaphoreType.DMA,
        pltpu.SemaphoreType.DMA,
    ],
)
def _sc_edges(tbl_hbm, adj_hbm, out_hbm, tbl_v, tbl_spm,
              src_v0, src_v1, tgt_v0, tgt_v1, out_v0, out_v1,
              sin0, sin1, sout0, sout1):
    wid = lax.axis_index("s") * _NC + lax.axis_index("c")

    srcs = (src_v0, src_v1)
    tgts = (tgt_v0, tgt_v1)
    outs = (out_v0, out_v1)
    sins = (sin0, sin1)
    souts = (sout0, sout1)

    def task_id(k):
        return wid + _NW * k

    def task_valid(k):
        return task_id(k) < _NTASK

    def start_in(k, b):
        off = task_id(k) * _EPT
        pltpu.async_copy(adj_hbm.at[0, pl.ds(off, _EPT)], srcs[b], sins[b])
        pltpu.async_copy(adj_hbm.at[1, pl.ds(off, _EPT)], tgts[b], sins[b])

    def wait_in(k, b):
        off = task_id(k) * _EPT
        pltpu.make_async_copy(adj_hbm.at[0, pl.ds(off, _EPT)], srcs[b],
                              sins[b]).wait()
        pltpu.make_async_copy(adj_hbm.at[1, pl.ds(off, _EPT)], tgts[b],
                              sins[b]).wait()

    def start_out(k, b):
        off = task_id(k) * _OPT
        pltpu.async_copy(outs[b], out_hbm.at[pl.ds(off, _OPT)], souts[b])

    def wait_out(k, b):
        off = task_id(k) * _OPT
        pltpu.make_async_copy(outs[b], out_hbm.at[pl.ds(off, _OPT)],
                              souts[b]).wait()

    def compute(k, b):
        src_v, tgt_v, out_v = srcs[b], tgts[b], outs[b]

        @plsc.parallel_loop(0, _MITER, unroll=4)
        def body(m):
            s = src_v[pl.ds(m * 16, 16)]
            t = tgt_v[pl.ds(m * 16, 16)]
            bs = (lax.shift_right_logical(s, 7) * (2 * _H * _NODE_BLK)
                  + lax.bitwise_and(s, _NODE_BLK - 1))
            bt = (lax.shift_right_logical(t, 7) * (2 * _H * _NODE_BLK)
                  + lax.bitwise_and(t, _NODE_BLK - 1))
            obase = (m // 8) * (_H * _BLK) + (m % 8) * 16
            for h in range(_H):
                u = plsc.load_gather(tbl_v, [bs + (h * _NODE_BLK)])
                v = plsc.load_gather(tbl_v, [bt + ((_H + h) * _NODE_BLK)])
                out_v[pl.ds(obase + h * _BLK, 16)] = jnp.maximum(u + v, 0.0)

    start_in(0, 0)
    _TBLW = _NTBLK * 2 * _H * _NODE_BLK
    _SEG = _TBLW // _NS
    sid = lax.axis_index("s")
    pltpu.sync_copy(tbl_hbm.at[pl.ds(sid * _SEG, _SEG)],
                    out_v0.at[pl.ds(0, _SEG)])
    pltpu.sync_copy(out_v0.at[pl.ds(0, _SEG)],
                    tbl_spm.at[pl.ds(sid * _SEG, _SEG)])
    plsc.subcore_barrier()
    pltpu.sync_copy(tbl_spm, tbl_v)

    def pair(kk, carry):
        k0 = 2 * kk
        k1 = k0 + 1

        @pl.when(task_valid(k1))
        def _():
            start_in(k1, 1)
        wait_in(k0, 0)

        @pl.when(k0 >= 2)
        def _():
            wait_out(k0 - 2, 0)
        compute(k0, 0)
        start_out(k0, 0)

        @pl.when(task_valid(k0 + 2))
        def _():
            start_in(k0 + 2, 0)

        @pl.when(task_valid(k1))
        def _():
            wait_in(k1, 1)

            @pl.when(k1 >= 3)
            def _():
                wait_out(k1 - 2, 1)
            compute(k1, 1)
            start_out(k1, 1)
        return carry

    lax.fori_loop(0, _KMAX // 2, pair, 0)
    wait_out(_KMAX - 2, 0)

    @pl.when(task_valid(_KMAX - 1))
    def _():
        wait_out(_KMAX - 1, 1)


def kernel(x_0, adjacency_0, att_parameter):
    adj = adjacency_0.astype(jnp.int32)
    p = _node_projection(x_0, att_parameter)
    tbl = p.reshape(2 * _H, _NTBLK, _NODE_BLK).transpose(1, 0, 2).reshape(-1)
    out_flat = _sc_edges(tbl, adj)
    out = out_flat.reshape(_NBLK, _H, _BLK).transpose(0, 2, 1).reshape(_E, _H)
    return out
